# baseline (speedup 1.0000x reference)
import functools

import jax
import jax.numpy as jnp
from jax import lax
from jax.experimental import pallas as pl
from jax.experimental.pallas import tpu as pltpu, tpu_sc as plsc

_ROWS = 8192
_DIM = 1024
_WORDS = _ROWS * _DIM
_NC = 2
_NS = 16
_NW = _NC * _NS
_W_PER = _WORDS // _NW
_CMAX = 65408
_NBUF = 2

_CHUNKS = [_CMAX] * (_W_PER // _CMAX)
if _W_PER % _CMAX:
    _CHUNKS.append(_W_PER % _CMAX)
_OFFS = [sum(_CHUNKS[:i]) for i in range(len(_CHUNKS))]
_NCHUNK = len(_CHUNKS)

_mesh = plsc.VectorSubcoreMesh(core_axis_name="c", subcore_axis_name="s")


@functools.partial(
    pl.kernel,
    mesh=_mesh,
    out_type=jax.ShapeDtypeStruct((_WORDS,), jnp.float32),
    scratch_types=(
        [pltpu.VMEM((_NBUF, _CMAX), jnp.float32)]
        + [pltpu.SemaphoreType.DMA] * (2 * _NBUF)
    ),
)
def _copy_table(table_hbm, out_hbm, buf, *sems):
    wid = lax.axis_index("s") * _NC + lax.axis_index("c")
    base = wid * _W_PER
    s_in = sems[:_NBUF]
    s_out = sems[_NBUF:]

    def cp_in(g, b):
        n = _CHUNKS[g]
        return pltpu.make_async_copy(
            table_hbm.at[pl.ds(base + _OFFS[g], n)], buf.at[b, pl.ds(0, n)],
            s_in[b])

    def cp_out(g, b):
        n = _CHUNKS[g]
        return pltpu.make_async_copy(
            buf.at[b, pl.ds(0, n)], out_hbm.at[pl.ds(base + _OFFS[g], n)],
            s_out[b])

    for g in range(_NBUF):
        cp_in(g, g).start()
    for g in range(_NCHUNK):
        b = g % _NBUF
        cp_in(g, b).wait()
        cp_out(g, b).start()
        if g + _NBUF < _NCHUNK:
            cp_out(g, b).wait()
            cp_in(g + _NBUF, b).start()
    for g in range(max(_NCHUNK - _NBUF, 0), _NCHUNK):
        cp_out(g, g % _NBUF).wait()


def kernel(x, table):
    return _copy_table(table.reshape(-1)).reshape(1, _ROWS, _DIM)


# speedup vs baseline: 2.4253x; 2.4253x over previous
import functools

import jax
import jax.numpy as jnp
from jax import lax
from jax.experimental import pallas as pl
from jax.experimental.pallas import tpu as pltpu, tpu_sc as plsc

_ROWS = 8192
_DIM = 1024
_NC = 2
_NS = 16
_NW = _NC * _NS
_ROWS_PER_W = _ROWS // _NW
_C = 32
_NCHUNK = _ROWS_PER_W // _C
_NBUF = 3

_mesh = plsc.VectorSubcoreMesh(core_axis_name="c", subcore_axis_name="s")


@functools.partial(
    pl.kernel,
    mesh=_mesh,
    out_type=jax.ShapeDtypeStruct((_ROWS, _DIM), jnp.float32),
    scratch_types=(
        [pltpu.VMEM((_NBUF, _C, _DIM), jnp.float32)]
        + [pltpu.SemaphoreType.DMA] * (2 * _NBUF)
    ),
)
def _copy_table(table_hbm, out_hbm, buf, *sems):
    wid = lax.axis_index("s") * _NC + lax.axis_index("c")
    base = wid * _ROWS_PER_W
    s_in = sems[:_NBUF]
    s_out = sems[_NBUF:]

    def cp_in(g, b):
        return pltpu.make_async_copy(
            table_hbm.at[pl.ds(base + g * _C, _C)], buf.at[b], s_in[b])

    def cp_out(g, b):
        return pltpu.make_async_copy(
            buf.at[b], out_hbm.at[pl.ds(base + g * _C, _C)], s_out[b])

    for g in range(min(_NBUF, _NCHUNK)):
        cp_in(g, g).start()
    out_pending = []
    for g in range(_NCHUNK):
        cp_in(g, g % _NBUF).wait()
        cp_out(g, g % _NBUF).start()
        out_pending.append(g)
        p = g - 1
        if p >= 0 and p + _NBUF < _NCHUNK:
            cp_out(p, p % _NBUF).wait()
            out_pending.remove(p)
            cp_in(p + _NBUF, p % _NBUF).start()
    for g in out_pending:
        cp_out(g, g % _NBUF).wait()


def kernel(x, table):
    return _copy_table(table)[None]


# speedup vs baseline: 2.5122x; 1.0358x over previous
import functools

import jax
import jax.numpy as jnp
from jax import lax
from jax.experimental import pallas as pl
from jax.experimental.pallas import tpu as pltpu, tpu_sc as plsc

_ROWS = 8192
_DIM = 1024
_NC = 2
_NS = 16
_NW = _NC * _NS
_ROWS_PER_W = _ROWS // _NW
_C = 56
_NBUF = 2

_CHUNKS = [_C] * (_ROWS_PER_W // _C)
if _ROWS_PER_W % _C:
    _CHUNKS.append(_ROWS_PER_W % _C)
_OFFS = [sum(_CHUNKS[:i]) for i in range(len(_CHUNKS))]
_NCHUNK = len(_CHUNKS)

_mesh = plsc.VectorSubcoreMesh(core_axis_name="c", subcore_axis_name="s")


@functools.partial(
    pl.kernel,
    mesh=_mesh,
    out_type=jax.ShapeDtypeStruct((_ROWS, _DIM), jnp.float32),
    scratch_types=(
        [pltpu.VMEM((_NBUF, _C, _DIM), jnp.float32)]
        + [pltpu.SemaphoreType.DMA] * (2 * _NBUF)
    ),
)
def _copy_table(table_hbm, out_hbm, buf, *sems):
    wid = lax.axis_index("s") * _NC + lax.axis_index("c")
    base = wid * _ROWS_PER_W
    s_in = sems[:_NBUF]
    s_out = sems[_NBUF:]

    def cp_in(g, b):
        n = _CHUNKS[g]
        return pltpu.make_async_copy(
            table_hbm.at[pl.ds(base + _OFFS[g], n)],
            buf.at[b, pl.ds(0, n)], s_in[b])

    def cp_out(g, b):
        n = _CHUNKS[g]
        return pltpu.make_async_copy(
            buf.at[b, pl.ds(0, n)],
            out_hbm.at[pl.ds(base + _OFFS[g], n)], s_out[b])

    for g in range(min(_NBUF, _NCHUNK)):
        cp_in(g, g).start()
    for g in range(_NCHUNK):
        b = g % _NBUF
        cp_in(g, b).wait()
        cp_out(g, b).start()
        if g + _NBUF < _NCHUNK:
            cp_out(g, b).wait()
            cp_in(g + _NBUF, b).start()
    for g in range(max(_NCHUNK - _NBUF, 0), _NCHUNK):
        cp_out(g, g % _NBUF).wait()


def kernel(x, table):
    return _copy_table(table)[None]


# speedup vs baseline: 2.5267x; 1.0058x over previous
import functools

import jax
import jax.numpy as jnp
from jax import lax
from jax.experimental import pallas as pl
from jax.experimental.pallas import tpu as pltpu, tpu_sc as plsc

_ROWS = 8192
_DIM = 1024
_NC = 2
_NS = 16
_NW = _NC * _NS
_ROWS_PER_W = _ROWS // _NW
_C = 32
_H = _C // 2
_NCHUNK = _ROWS_PER_W // _C
_NBUF = 3

_mesh = plsc.VectorSubcoreMesh(core_axis_name="c", subcore_axis_name="s")


@functools.partial(
    pl.kernel,
    mesh=_mesh,
    out_type=jax.ShapeDtypeStruct((_ROWS, _DIM), jnp.float32),
    scratch_types=(
        [pltpu.VMEM((_NBUF, _C, _DIM), jnp.float32)]
        + [pltpu.SemaphoreType.DMA] * (4 * _NBUF)
    ),
)
def _copy_table(table_hbm, out_hbm, buf, *sems):
    wid = lax.axis_index("s") * _NC + lax.axis_index("c")
    base = wid * _ROWS_PER_W
    s_in = sems[: 2 * _NBUF]
    s_out = sems[2 * _NBUF:]

    def cp_in(g, b, h):
        return pltpu.make_async_copy(
            table_hbm.at[pl.ds(base + g * _C + h * _H, _H)],
            buf.at[b, pl.ds(h * _H, _H)], s_in[2 * b + h])

    def cp_out(g, b, h):
        return pltpu.make_async_copy(
            buf.at[b, pl.ds(h * _H, _H)],
            out_hbm.at[pl.ds(base + g * _C + h * _H, _H)], s_out[2 * b + h])

    for g in range(_NBUF):
        cp_in(g, g, 0).start()
        cp_in(g, g, 1).start()
    for g in range(_NCHUNK):
        b = g % _NBUF
        cp_in(g, b, 0).wait()
        cp_in(g, b, 1).wait()
        cp_out(g, b, 0).start()
        cp_out(g, b, 1).start()
        if g + _NBUF < _NCHUNK:
            cp_out(g, b, 0).wait()
            cp_out(g, b, 1).wait()
            cp_in(g + _NBUF, b, 0).start()
            cp_in(g + _NBUF, b, 1).start()
    for g in range(_NCHUNK - _NBUF, _NCHUNK):
        cp_out(g, g % _NBUF, 0).wait()
        cp_out(g, g % _NBUF, 1).wait()


def kernel(x, table):
    return _copy_table(table)[None]
